# Initial kernel scaffold; baseline (speedup 1.0000x reference)
import jax
import jax.numpy as jnp
from jax.experimental import pallas as pl


def kernel(input_ids, token_table, pos_table):
    raise NotImplementedError("write your pallas kernel here")


# speedup vs baseline: 1.0372x; 1.0372x over previous
import functools

import jax
import jax.numpy as jnp
from jax import lax
from jax.experimental import pallas as pl
from jax.experimental.pallas import tpu as pltpu
from jax.experimental.pallas import tpu_sc as plsc

_info = plsc.get_sparse_core_info()
_NC, _NS, _L = _info.num_cores, _info.num_subcores, _info.num_lanes
_NW = _NC * _NS


@functools.lru_cache(maxsize=None)
def _build(batch, seq_len, vocab, dim):
    total = batch * seq_len
    rows_per_w = total // _NW
    assert total % _NW == 0 and rows_per_w % 8 == 0 and dim % _L == 0
    assert seq_len % rows_per_w == 0
    w_per_batch = seq_len // rows_per_w

    def body(idx_hbm, pos_hbm, table_hbm, out_hbm, idx_v, acc_v, pos_v, sem):
        wid = lax.axis_index("s") * _NC + lax.axis_index("c")
        base = wid * rows_per_w
        pos_base = lax.rem(wid, w_per_batch) * rows_per_w
        pltpu.sync_copy(idx_hbm.at[pl.ds(base, rows_per_w)], idx_v)
        pltpu.sync_copy(pos_hbm.at[pl.ds(pos_base, rows_per_w), :], pos_v)
        pltpu.async_copy(table_hbm.at[idx_v], acc_v, sem).wait()

        def row(i, carry):
            for j in range(dim // _L):
                sl = pl.ds(j * _L, _L)
                acc_v[i, sl] = acc_v[i, sl] + pos_v[i, sl]
            return carry

        lax.fori_loop(0, rows_per_w, row, 0)
        pltpu.sync_copy(acc_v, out_hbm.at[pl.ds(base, rows_per_w), :])

    mesh = plsc.VectorSubcoreMesh(core_axis_name="c", subcore_axis_name="s")
    kern = pl.kernel(
        body,
        mesh=mesh,
        out_type=jax.ShapeDtypeStruct((total, dim), jnp.float32),
        scratch_types=[
            pltpu.VMEM((rows_per_w,), jnp.int32),
            pltpu.VMEM((rows_per_w, dim), jnp.float32),
            pltpu.VMEM((rows_per_w, dim), jnp.float32),
            pltpu.SemaphoreType.DMA,
        ],
    )

    @jax.jit
    def run(input_ids, token_table, pos_table):
        idx_flat = input_ids.reshape(-1).astype(jnp.int32)
        out = kern(idx_flat, pos_table, token_table)
        return out.reshape(batch, seq_len, dim)

    return run


def kernel(input_ids, token_table, pos_table):
    batch, seq_len = input_ids.shape
    vocab, dim = token_table.shape
    return _build(batch, seq_len, vocab, dim)(input_ids, token_table, pos_table)


# speedup vs baseline: 1.1612x; 1.1196x over previous
import functools

import jax
import jax.numpy as jnp
from jax import lax
from jax.experimental import pallas as pl
from jax.experimental.pallas import tpu as pltpu
from jax.experimental.pallas import tpu_sc as plsc

_info = plsc.get_sparse_core_info()
_NC, _NS, _L = _info.num_cores, _info.num_subcores, _info.num_lanes
_NW = _NC * _NS


@functools.lru_cache(maxsize=None)
def _build(batch, seq_len, vocab, dim):
    total = batch * seq_len
    rows_per_w = total // _NW
    assert total % _NW == 0 and rows_per_w % 8 == 0 and dim % _L == 0
    assert seq_len % rows_per_w == 0
    w_per_batch = seq_len // rows_per_w

    def body(idx_hbm, pos_hbm, table_hbm, out_hbm, idx_v, acc_v, sem0, sem1):
        wid = lax.axis_index("s") * _NC + lax.axis_index("c")
        base = wid * rows_per_w
        pos_base = lax.rem(wid, w_per_batch) * rows_per_w
        cp_idx = pltpu.async_copy(idx_hbm.at[pl.ds(base, rows_per_w)], idx_v, sem0)
        cp_pos = pltpu.async_copy(
            pos_hbm.at[pl.ds(pos_base, rows_per_w), :], acc_v, sem1)
        cp_idx.wait()
        cp_pos.wait()
        pltpu.async_copy(table_hbm.at[idx_v], acc_v, sem0, add=True).wait()
        pltpu.sync_copy(acc_v, out_hbm.at[pl.ds(base, rows_per_w), :])

    mesh = plsc.VectorSubcoreMesh(core_axis_name="c", subcore_axis_name="s")
    kern = pl.kernel(
        body,
        mesh=mesh,
        out_type=jax.ShapeDtypeStruct((total, dim), jnp.float32),
        scratch_types=[
            pltpu.VMEM((rows_per_w,), jnp.int32),
            pltpu.VMEM((rows_per_w, dim), jnp.float32),
            pltpu.SemaphoreType.DMA,
            pltpu.SemaphoreType.DMA,
        ],
    )

    @jax.jit
    def run(input_ids, token_table, pos_table):
        idx_flat = input_ids.reshape(-1).astype(jnp.int32)
        out = kern(idx_flat, pos_table, token_table)
        return out.reshape(batch, seq_len, dim)

    return run


def kernel(input_ids, token_table, pos_table):
    batch, seq_len = input_ids.shape
    vocab, dim = token_table.shape
    return _build(batch, seq_len, vocab, dim)(input_ids, token_table, pos_table)
